# Initial kernel scaffold; baseline (speedup 1.0000x reference)
import jax
import jax.numpy as jnp
from jax.experimental import pallas as pl


def kernel(x, edge_index, edge_type, bases0, comp0, root0, bias0, ln_g0, ln_b0, bases1, comp1, root1, bias1, ln_g1, ln_b1, bases2, comp2, root2, bias2, ln_g2, ln_b2, edge_emb, attn_w1, attn_b1, attn_w2, attn_b2):
    raise NotImplementedError("write your pallas kernel here")


# speedup vs baseline: 10.0166x; 10.0166x over previous
import functools

import jax
import jax.numpy as jnp
from jax import lax
from jax.experimental import pallas as pl
from jax.experimental.pallas import tpu as pltpu
from jax.experimental.pallas import tpu_sc as plsc

N = 10000
E = 320000
R = 4
DH = 256
NB = 8
SEG = N * R
C0 = 128 // 16
C12 = 256 // 16

NTILES = 16
NCORES = 2
EP = E // NTILES
BB = 128
NBATCH = 160
RING = 8
ROUNDS = NBATCH // RING
CHUNK = 16
ACC_ROWS = SEG + NTILES
ZROWS = ACC_ROWS // NTILES
DROWS = SEG // NTILES
BN = 1000
PADLANES = 128


def _edge_prep_body(src_ref, dst_ref, et_ref, seg_ref, gb4_ref, gb8_ref):
    pad = NBATCH * BB - EP
    src = src_ref[...]
    seg = et_ref[...] * N + dst_ref[...]
    pad_seg = SEG + lax.broadcasted_iota(jnp.int32, (NTILES, pad), 0)
    zpad = jnp.zeros((NTILES, pad), jnp.int32)
    seg_ref[...] = jnp.concatenate([seg, pad_seg], axis=1)
    gb4_ref[...] = jnp.concatenate([src * C0, zpad], axis=1)
    gb8_ref[...] = jnp.concatenate([src * C12, zpad], axis=1)


def _edge_prep(src, dst, et):
    out = jax.ShapeDtypeStruct((NTILES, NBATCH * BB), jnp.int32)
    return pl.pallas_call(
        _edge_prep_body, out_shape=[out, out, out],
    )(src.reshape(NTILES, EP), dst.reshape(NTILES, EP), et.reshape(NTILES, EP))


def _make_sc_scatter(C, with_counts):
    C_sc = C // NCORES
    mesh = plsc.VectorSubcoreMesh(core_axis_name="c", subcore_axis_name="s")
    out_type = [jax.ShapeDtypeStruct((SEG, C * CHUNK), jnp.float32)]
    if with_counts:
        out_type.append(jax.ShapeDtypeStruct((SEG, CHUNK), jnp.float32))
    scratch = [
        pltpu.VMEM((NBATCH, BB), jnp.int32),
        pltpu.VMEM((NBATCH, BB), jnp.int32),
        pltpu.VMEM((NBATCH, BB), jnp.int32),
        pltpu.VMEM((RING, BB, CHUNK), jnp.float32),
        pltpu.VMEM((BB, CHUNK), jnp.float32),
        pltpu.VMEM_SHARED((ACC_ROWS, CHUNK), jnp.float32),
        pltpu.SemaphoreType.DMA,
    ]

    @functools.partial(pl.kernel, mesh=mesh, out_type=out_type,
                       scratch_types=scratch,
                       compiler_params=pltpu.CompilerParams(
                           use_tc_tiling_on_sc=False))
    def sck(xr_hbm, seg_hbm, gb_hbm, zeros_hbm, ones_hbm, acc_hbm, *rest):
        if with_counts:
            cnt_hbm, seg2d, gb2d, gidx2d, ring, zbuf, acc_sp, sem = rest
        else:
            seg2d, gb2d, gidx2d, ring, zbuf, acc_sp, sem = rest
        core = lax.axis_index("c")
        s = lax.axis_index("s")

        pltpu.sync_copy(seg_hbm.at[s], seg2d)
        pltpu.sync_copy(gb_hbm.at[s], gb2d)
        pltpu.sync_copy(zeros_hbm, zbuf)

        def zero_acc():
            base = s * ZROWS
            def zbody(k, carry):
                pltpu.sync_copy(zbuf, acc_sp.at[pl.ds(base + k * BB, BB)])
                return carry
            nfull = ZROWS // BB
            lax.fori_loop(0, nfull, zbody, 0)
            tail = ZROWS - nfull * BB
            pltpu.sync_copy(zbuf.at[pl.ds(0, tail)],
                            acc_sp.at[pl.ds(base + nfull * BB, tail)])

        for cc in range(C_sc):
            c = core * C_sc + cc

            def gi_body(j, carry):
                for p in range(BB // 16):
                    sl = pl.ds(p * 16, 16)
                    gidx2d[j, sl] = gb2d[j, sl] + c
                return carry
            lax.fori_loop(0, NBATCH, gi_body, 0)

            zero_acc()
            plsc.subcore_barrier()

            for b in range(RING):
                pltpu.make_async_copy(
                    xr_hbm.at[gidx2d.at[b]], ring.at[b], sem).start()

            def round_body(rr, carry):
                for b in range(RING):
                    j = rr * RING + b
                    pltpu.make_async_copy(
                        xr_hbm.at[gidx2d.at[0]], ring.at[b], sem).wait()
                    pltpu.sync_copy(ring.at[b], acc_sp.at[seg2d.at[j]],
                                    add=True)
                    jn = j + RING
                    @pl.when(jn < NBATCH)
                    def _():
                        pltpu.make_async_copy(
                            xr_hbm.at[gidx2d.at[jn]], ring.at[b], sem).start()
                return carry
            lax.fori_loop(0, ROUNDS, round_body, 0)
            plsc.subcore_barrier()

            pltpu.sync_copy(
                acc_sp.at[pl.ds(s * DROWS, DROWS)],
                acc_hbm.at[pl.ds(s * DROWS, DROWS),
                           pl.ds(c * CHUNK, CHUNK)])
            plsc.subcore_barrier()

        if with_counts:
            @pl.when(core == 0)
            def _():
                zero_acc()
                plsc.subcore_barrier()
                pltpu.sync_copy(ones_hbm, ring.at[0])
                def cbody(j, carry):
                    pltpu.sync_copy(ring.at[0], acc_sp.at[seg2d.at[j]],
                                    add=True)
                    return carry
                lax.fori_loop(0, NBATCH, cbody, 0)
                plsc.subcore_barrier()
                pltpu.sync_copy(acc_sp.at[pl.ds(s * DROWS, DROWS)],
                                cnt_hbm.at[pl.ds(s * DROWS, DROWS)])

    return sck


_sc_cache = {}


def _sc_scatter(C, with_counts):
    key = (C, with_counts)
    if key not in _sc_cache:
        _sc_cache[key] = _make_sc_scatter(C, with_counts)
    return _sc_cache[key]


def _wcomb_body(comp_ref, basesf_ref, out_ref):
    out_ref[...] = jnp.dot(comp_ref[...], basesf_ref[...],
                           preferred_element_type=jnp.float32)


def _wcomb(comp, basesf):
    L = basesf.shape[1]
    wblk = 2048
    return pl.pallas_call(
        _wcomb_body,
        grid=(L // wblk,),
        in_specs=[pl.BlockSpec((R, NB), lambda i: (0, 0)),
                  pl.BlockSpec((NB, wblk), lambda i: (0, i))],
        out_specs=pl.BlockSpec((R, wblk), lambda i: (0, i)),
        out_shape=jax.ShapeDtypeStruct((R, L), jnp.float32),
    )(comp, basesf)


def _layer_body(res, acc_ref, cnt_ref, h_ref, w_ref, root_ref, bias_ref,
                g_ref, b_ref, out_ref):
    cnt = cnt_ref[...][:, :, :1]
    recip = 1.0 / jnp.maximum(cnt, 1.0)
    m = acc_ref[...] * recip
    y = jnp.dot(h_ref[...], root_ref[...], preferred_element_type=jnp.float32)
    for r in range(R):
        y = y + jnp.dot(m[r], w_ref[r], preferred_element_type=jnp.float32)
    y = y + bias_ref[...]
    mu = jnp.mean(y, axis=-1, keepdims=True)
    d = y - mu
    var = jnp.mean(d * d, axis=-1, keepdims=True)
    y = d * lax.rsqrt(var + 1e-5) * g_ref[...] + b_ref[...]
    y = jnp.where(y > 0, y, jnp.exp(y) - 1.0)
    if res:
        y = y + h_ref[...]
    out_ref[...] = y


def _layer(res, acc3, cnt3, h, w3, root, bias, g, b):
    di = h.shape[1]
    vec = pl.BlockSpec((1, DH), lambda i: (0, 0))
    return pl.pallas_call(
        functools.partial(_layer_body, res),
        grid=(N // BN,),
        in_specs=[pl.BlockSpec((R, BN, di), lambda i: (0, i, 0)),
                  pl.BlockSpec((R, BN, CHUNK), lambda i: (0, i, 0)),
                  pl.BlockSpec((BN, di), lambda i: (i, 0)),
                  pl.BlockSpec((R, di, DH), lambda i: (0, 0, 0)),
                  pl.BlockSpec((di, DH), lambda i: (0, 0)),
                  vec, vec, vec],
        out_specs=pl.BlockSpec((BN, DH), lambda i: (i, 0)),
        out_shape=jax.ShapeDtypeStruct((N, DH), jnp.float32),
    )(acc3, cnt3, h, w3, root, bias, g, b)


def _final_body(h_ref, ee_ref, w1_ref, b1_ref, w2_ref, b2_ref, out_ref, sacc):
    i = pl.program_id(0)

    @pl.when(i == 0)
    def _():
        sacc[...] = jnp.zeros_like(sacc)

    sacc[0:1, :] = sacc[0:1, :] + jnp.sum(h_ref[...], axis=0, keepdims=True)

    @pl.when(i == N // BN - 1)
    def _():
        gmean = sacc[0:1, :] * (1.0 / N)
        rep = jnp.broadcast_to(gmean, (8, DH))
        att = jnp.concatenate([rep, ee_ref[...]], axis=1)
        a = jnp.dot(att, w1_ref[...], preferred_element_type=jnp.float32)
        a = jnp.maximum(a + b1_ref[...], 0.0)
        logits = jnp.dot(a, w2_ref[...], preferred_element_type=jnp.float32)
        logits = logits + b2_ref[...]
        lane = lax.broadcasted_iota(jnp.int32, (8, PADLANES), 1)
        ml = jnp.where(lane < R, logits, jnp.full_like(logits, -1e30))
        mx = jnp.max(ml, axis=-1, keepdims=True)
        ex = jnp.exp(ml - mx)
        ex = jnp.where(lane < R, ex, 0.0)
        out_ref[...] = ex / jnp.sum(ex, axis=-1, keepdims=True)


def _final(h, ee_pad, w1, b1, w2_pad, b2_pad):
    return pl.pallas_call(
        _final_body,
        grid=(N // BN,),
        in_specs=[pl.BlockSpec((BN, DH), lambda i: (i, 0)),
                  pl.BlockSpec((8, DH), lambda i: (0, 0)),
                  pl.BlockSpec((2 * DH, DH), lambda i: (0, 0)),
                  pl.BlockSpec((1, DH), lambda i: (0, 0)),
                  pl.BlockSpec((DH, PADLANES), lambda i: (0, 0)),
                  pl.BlockSpec((1, PADLANES), lambda i: (0, 0))],
        out_specs=pl.BlockSpec((8, PADLANES), lambda i: (0, 0)),
        out_shape=jax.ShapeDtypeStruct((8, PADLANES), jnp.float32),
        scratch_shapes=[pltpu.VMEM((8, DH), jnp.float32)],
    )(h, ee_pad, w1, b1, w2_pad, b2_pad)


def kernel(x, edge_index, edge_type, bases0, comp0, root0, bias0, ln_g0,
           ln_b0, bases1, comp1, root1, bias1, ln_g1, ln_b1, bases2, comp2,
           root2, bias2, ln_g2, ln_b2, edge_emb, attn_w1, attn_b1, attn_w2,
           attn_b2):
    src = edge_index[0]
    dst = edge_index[1]
    seg3, gb4, gb8 = _edge_prep(src, dst, edge_type)
    seg3 = seg3.reshape(NTILES, NBATCH, BB)
    gb4 = gb4.reshape(NTILES, NBATCH, BB)
    gb8 = gb8.reshape(NTILES, NBATCH, BB)
    zeros = jnp.zeros((BB, CHUNK), jnp.float32)
    ones = jnp.ones((BB, CHUNK), jnp.float32)

    params = [
        (bases0, comp0, root0, bias0, ln_g0, ln_b0),
        (bases1, comp1, root1, bias1, ln_g1, ln_b1),
        (bases2, comp2, root2, bias2, ln_g2, ln_b2),
    ]
    h = x
    cnt3 = None
    for l, (ba, co, ro, bi, g, b) in enumerate(params):
        di = h.shape[1]
        C = di // CHUNK
        xr = h.reshape(N * C, CHUNK)
        if l == 0:
            acc, cntw = _sc_scatter(C0, True)(xr, seg3, gb4, zeros, ones)
            cnt3 = cntw.reshape(R, N, CHUNK)
        else:
            out = _sc_scatter(C12, False)(xr, seg3, gb8, zeros, ones)
            acc = out[0] if isinstance(out, (list, tuple)) else out
        w3 = _wcomb(co, ba.reshape(NB, di * DH)).reshape(R, di, DH)
        h = _layer(l > 0, acc.reshape(R, N, di), cnt3, h, w3, ro,
                   bi.reshape(1, DH), g.reshape(1, DH), b.reshape(1, DH))

    ee_pad = jnp.concatenate(
        [edge_emb, jnp.zeros((8 - R, DH), jnp.float32)], axis=0)
    w2_pad = jnp.concatenate(
        [attn_w2, jnp.zeros((DH, PADLANES - R), jnp.float32)], axis=1)
    b2_pad = jnp.concatenate(
        [attn_b2, jnp.zeros((PADLANES - R,), jnp.float32)]).reshape(1, PADLANES)
    scores = _final(h, ee_pad, attn_w1, attn_b1.reshape(1, DH),
                    w2_pad, b2_pad)[:R, :R]
    return h, scores


# speedup vs baseline: 11.8587x; 1.1839x over previous
import functools

import jax
import jax.numpy as jnp
from jax import lax
from jax.experimental import pallas as pl
from jax.experimental.pallas import tpu as pltpu
from jax.experimental.pallas import tpu_sc as plsc

N = 10000
E = 320000
R = 4
DH = 256
NB = 8
SEG = N * R
CHUNK = 32
C0 = 128 // CHUNK
C12 = 256 // CHUNK

NTILES = 16
NCORES = 2
EP = E // NTILES
BB = 128
NBATCH = 160
RING = 8
ROUNDS = NBATCH // RING
NPASS = 4
PB = NBATCH // NPASS
ACC_ROWS = SEG + NTILES
ZROWS = ACC_ROWS // NTILES
DROWS = SEG // NTILES
BN = 1000
PADLANES = 128


def _edge_prep_body(src_ref, dst_ref, et_ref, seg_ref, gb4_ref, gb8_ref):
    pad = NBATCH * BB - EP
    src = src_ref[...]
    seg = et_ref[...] * N + dst_ref[...]
    pad_seg = SEG + lax.broadcasted_iota(jnp.int32, (NTILES, pad), 0)
    zpad = jnp.zeros((NTILES, pad), jnp.int32)
    seg_ref[...] = jnp.concatenate([seg, pad_seg], axis=1)
    gb4_ref[...] = jnp.concatenate([src * C0, zpad], axis=1)
    gb8_ref[...] = jnp.concatenate([src * C12, zpad], axis=1)


def _edge_prep(src, dst, et):
    out = jax.ShapeDtypeStruct((NTILES, NBATCH * BB), jnp.int32)
    return pl.pallas_call(
        _edge_prep_body, out_shape=[out, out, out],
    )(src.reshape(NTILES, EP), dst.reshape(NTILES, EP), et.reshape(NTILES, EP))


def _make_sc_scatter(C, with_counts):
    C_sc = C // NCORES
    mesh = plsc.VectorSubcoreMesh(core_axis_name="c", subcore_axis_name="s")
    out_type = [jax.ShapeDtypeStruct((SEG, C * CHUNK), jnp.float32)]
    if with_counts:
        out_type.append(jax.ShapeDtypeStruct((SEG, CHUNK), jnp.float32))
    scratch = [
        pltpu.VMEM((PB, BB), jnp.int32),
        pltpu.VMEM((PB, BB), jnp.int32),
        pltpu.VMEM((RING, BB, CHUNK), jnp.float32),
        pltpu.VMEM((BB, CHUNK), jnp.float32),
        pltpu.VMEM_SHARED((ACC_ROWS, CHUNK), jnp.float32),
        pltpu.SemaphoreType.DMA,
    ]

    @functools.partial(pl.kernel, mesh=mesh, out_type=out_type,
                       scratch_types=scratch,
                       compiler_params=pltpu.CompilerParams(
                           use_tc_tiling_on_sc=False))
    def sck(xr_hbm, seg_hbm, gb_hbm, zeros_hbm, ones_hbm, acc_hbm, *rest):
        if with_counts:
            cnt_hbm, seg2d, gidx2d, ring, zbuf, acc_sp, sem = rest
        else:
            seg2d, gidx2d, ring, zbuf, acc_sp, sem = rest
        core = lax.axis_index("c")
        s = lax.axis_index("s")

        pltpu.sync_copy(zeros_hbm, zbuf)

        def zero_acc():
            base = s * ZROWS
            def zbody(k, carry):
                pltpu.sync_copy(zbuf, acc_sp.at[pl.ds(base + k * BB, BB)])
                return carry
            nfull = ZROWS // BB
            lax.fori_loop(0, nfull, zbody, 0)
            tail = ZROWS - nfull * BB
            pltpu.sync_copy(zbuf.at[pl.ds(0, tail)],
                            acc_sp.at[pl.ds(base + nfull * BB, tail)])

        for cc in range(C_sc):
            c = core * C_sc + cc

            zero_acc()
            plsc.subcore_barrier()

            for p in range(NPASS):
                pltpu.sync_copy(seg_hbm.at[s, pl.ds(p * PB, PB)], seg2d)
                pltpu.sync_copy(gb_hbm.at[s, pl.ds(p * PB, PB)], gidx2d)

                def gi_body(j, carry):
                    for g in range(BB // 16):
                        sl = pl.ds(g * 16, 16)
                        gidx2d[j, sl] = gidx2d[j, sl] + c
                    return carry
                lax.fori_loop(0, PB, gi_body, 0)

                for b in range(RING):
                    pltpu.make_async_copy(
                        xr_hbm.at[gidx2d.at[b]], ring.at[b], sem).start()

                def round_body(rr, carry):
                    for b in range(RING):
                        j = rr * RING + b
                        pltpu.make_async_copy(
                            xr_hbm.at[gidx2d.at[0]], ring.at[b], sem).wait()
                        pltpu.sync_copy(ring.at[b], acc_sp.at[seg2d.at[j]],
                                        add=True)
                        jn = j + RING
                        @pl.when(jn < PB)
                        def _():
                            pltpu.make_async_copy(
                                xr_hbm.at[gidx2d.at[jn]], ring.at[b],
                                sem).start()
                    return carry
                lax.fori_loop(0, PB // RING, round_body, 0)
            plsc.subcore_barrier()

            pltpu.sync_copy(
                acc_sp.at[pl.ds(s * DROWS, DROWS)],
                acc_hbm.at[pl.ds(s * DROWS, DROWS),
                           pl.ds(c * CHUNK, CHUNK)])
            plsc.subcore_barrier()

        if with_counts:
            @pl.when(core == 0)
            def _():
                zero_acc()
                plsc.subcore_barrier()
                pltpu.sync_copy(ones_hbm, ring.at[0])
                for p in range(NPASS):
                    pltpu.sync_copy(seg_hbm.at[s, pl.ds(p * PB, PB)], seg2d)
                    def cbody(j, carry):
                        pltpu.sync_copy(ring.at[0], acc_sp.at[seg2d.at[j]],
                                        add=True)
                        return carry
                    lax.fori_loop(0, PB, cbody, 0)
                plsc.subcore_barrier()
                pltpu.sync_copy(acc_sp.at[pl.ds(s * DROWS, DROWS)],
                                cnt_hbm.at[pl.ds(s * DROWS, DROWS)])

    return sck


_sc_cache = {}


def _sc_scatter(C, with_counts):
    key = (C, with_counts)
    if key not in _sc_cache:
        _sc_cache[key] = _make_sc_scatter(C, with_counts)
    return _sc_cache[key]


def _wcomb_body(comp_ref, basesf_ref, out_ref):
    out_ref[...] = jnp.dot(comp_ref[...], basesf_ref[...],
                           preferred_element_type=jnp.float32)


def _wcomb(comp, basesf):
    L = basesf.shape[1]
    wblk = 2048
    return pl.pallas_call(
        _wcomb_body,
        grid=(L // wblk,),
        in_specs=[pl.BlockSpec((R, NB), lambda i: (0, 0)),
                  pl.BlockSpec((NB, wblk), lambda i: (0, i))],
        out_specs=pl.BlockSpec((R, wblk), lambda i: (0, i)),
        out_shape=jax.ShapeDtypeStruct((R, L), jnp.float32),
    )(comp, basesf)


def _layer_body(res, acc_ref, cnt_ref, h_ref, w_ref, root_ref, bias_ref,
                g_ref, b_ref, out_ref):
    cnt = cnt_ref[...][:, :, :1]
    recip = 1.0 / jnp.maximum(cnt, 1.0)
    m = acc_ref[...] * recip
    y = jnp.dot(h_ref[...], root_ref[...], preferred_element_type=jnp.float32)
    for r in range(R):
        y = y + jnp.dot(m[r], w_ref[r], preferred_element_type=jnp.float32)
    y = y + bias_ref[...]
    mu = jnp.mean(y, axis=-1, keepdims=True)
    d = y - mu
    var = jnp.mean(d * d, axis=-1, keepdims=True)
    y = d * lax.rsqrt(var + 1e-5) * g_ref[...] + b_ref[...]
    y = jnp.where(y > 0, y, jnp.exp(y) - 1.0)
    if res:
        y = y + h_ref[...]
    out_ref[...] = y


def _layer(res, acc3, cnt3, h, w3, root, bias, g, b):
    di = h.shape[1]
    vec = pl.BlockSpec((1, DH), lambda i: (0, 0))
    return pl.pallas_call(
        functools.partial(_layer_body, res),
        grid=(N // BN,),
        in_specs=[pl.BlockSpec((R, BN, di), lambda i: (0, i, 0)),
                  pl.BlockSpec((R, BN, CHUNK), lambda i: (0, i, 0)),
                  pl.BlockSpec((BN, di), lambda i: (i, 0)),
                  pl.BlockSpec((R, di, DH), lambda i: (0, 0, 0)),
                  pl.BlockSpec((di, DH), lambda i: (0, 0)),
                  vec, vec, vec],
        out_specs=pl.BlockSpec((BN, DH), lambda i: (i, 0)),
        out_shape=jax.ShapeDtypeStruct((N, DH), jnp.float32),
    )(acc3, cnt3, h, w3, root, bias, g, b)


def _final_body(h_ref, ee_ref, w1_ref, b1_ref, w2_ref, b2_ref, out_ref, sacc):
    i = pl.program_id(0)

    @pl.when(i == 0)
    def _():
        sacc[...] = jnp.zeros_like(sacc)

    sacc[0:1, :] = sacc[0:1, :] + jnp.sum(h_ref[...], axis=0, keepdims=True)

    @pl.when(i == N // BN - 1)
    def _():
        gmean = sacc[0:1, :] * (1.0 / N)
        rep = jnp.broadcast_to(gmean, (8, DH))
        att = jnp.concatenate([rep, ee_ref[...]], axis=1)
        a = jnp.dot(att, w1_ref[...], preferred_element_type=jnp.float32)
        a = jnp.maximum(a + b1_ref[...], 0.0)
        logits = jnp.dot(a, w2_ref[...], preferred_element_type=jnp.float32)
        logits = logits + b2_ref[...]
        lane = lax.broadcasted_iota(jnp.int32, (8, PADLANES), 1)
        ml = jnp.where(lane < R, logits, jnp.full_like(logits, -1e30))
        mx = jnp.max(ml, axis=-1, keepdims=True)
        ex = jnp.exp(ml - mx)
        ex = jnp.where(lane < R, ex, 0.0)
        out_ref[...] = ex / jnp.sum(ex, axis=-1, keepdims=True)


def _final(h, ee_pad, w1, b1, w2_pad, b2_pad):
    return pl.pallas_call(
        _final_body,
        grid=(N // BN,),
        in_specs=[pl.BlockSpec((BN, DH), lambda i: (i, 0)),
                  pl.BlockSpec((8, DH), lambda i: (0, 0)),
                  pl.BlockSpec((2 * DH, DH), lambda i: (0, 0)),
                  pl.BlockSpec((1, DH), lambda i: (0, 0)),
                  pl.BlockSpec((DH, PADLANES), lambda i: (0, 0)),
                  pl.BlockSpec((1, PADLANES), lambda i: (0, 0))],
        out_specs=pl.BlockSpec((8, PADLANES), lambda i: (0, 0)),
        out_shape=jax.ShapeDtypeStruct((8, PADLANES), jnp.float32),
        scratch_shapes=[pltpu.VMEM((8, DH), jnp.float32)],
    )(h, ee_pad, w1, b1, w2_pad, b2_pad)


def kernel(x, edge_index, edge_type, bases0, comp0, root0, bias0, ln_g0,
           ln_b0, bases1, comp1, root1, bias1, ln_g1, ln_b1, bases2, comp2,
           root2, bias2, ln_g2, ln_b2, edge_emb, attn_w1, attn_b1, attn_w2,
           attn_b2):
    src = edge_index[0]
    dst = edge_index[1]
    seg3, gb4, gb8 = _edge_prep(src, dst, edge_type)
    seg3 = seg3.reshape(NTILES, NBATCH, BB)
    gb4 = gb4.reshape(NTILES, NBATCH, BB)
    gb8 = gb8.reshape(NTILES, NBATCH, BB)
    zeros = jnp.zeros((BB, CHUNK), jnp.float32)
    ones = jnp.ones((BB, CHUNK), jnp.float32)

    params = [
        (bases0, comp0, root0, bias0, ln_g0, ln_b0),
        (bases1, comp1, root1, bias1, ln_g1, ln_b1),
        (bases2, comp2, root2, bias2, ln_g2, ln_b2),
    ]
    h = x
    cnt3 = None
    for l, (ba, co, ro, bi, g, b) in enumerate(params):
        di = h.shape[1]
        C = di // CHUNK
        xr = h.reshape(N * C, CHUNK)
        if l == 0:
            acc, cntw = _sc_scatter(C0, True)(xr, seg3, gb4, zeros, ones)
            cnt3 = cntw.reshape(R, N, CHUNK)
        else:
            out = _sc_scatter(C12, False)(xr, seg3, gb8, zeros, ones)
            acc = out[0] if isinstance(out, (list, tuple)) else out
        w3 = _wcomb(co, ba.reshape(NB, di * DH)).reshape(R, di, DH)
        h = _layer(l > 0, acc.reshape(R, N, di), cnt3, h, w3, ro,
                   bi.reshape(1, DH), g.reshape(1, DH), b.reshape(1, DH))

    ee_pad = jnp.concatenate(
        [edge_emb, jnp.zeros((8 - R, DH), jnp.float32)], axis=0)
    w2_pad = jnp.concatenate(
        [attn_w2, jnp.zeros((DH, PADLANES - R), jnp.float32)], axis=1)
    b2_pad = jnp.concatenate(
        [attn_b2, jnp.zeros((PADLANES - R,), jnp.float32)]).reshape(1, PADLANES)
    scores = _final(h, ee_pad, attn_w1, attn_b1.reshape(1, DH),
                    w2_pad, b2_pad)[:R, :R]
    return h, scores


# speedup vs baseline: 11.9327x; 1.0062x over previous
import functools

import jax
import jax.numpy as jnp
from jax import lax
from jax.experimental import pallas as pl
from jax.experimental.pallas import tpu as pltpu
from jax.experimental.pallas import tpu_sc as plsc

N = 10000
E = 320000
R = 4
DH = 256
NB = 8
SEG = N * R
CHUNK = 32
C0 = 128 // CHUNK
C12 = 256 // CHUNK

NTILES = 16
NCORES = 2
EP = E // NTILES
BB = 128
NBATCH = 160
RING = 8
KS = 2
ROUNDS = NBATCH // RING
NPASS = 4
PB = NBATCH // NPASS
ACC_ROWS = SEG + NTILES
ZROWS = ACC_ROWS // NTILES
DROWS = SEG // NTILES
BN = 1000
PADLANES = 128


def _edge_prep_body(src_ref, dst_ref, et_ref, seg_ref, gb4_ref, gb8_ref):
    pad = NBATCH * BB - EP
    src = src_ref[...]
    seg = et_ref[...] * N + dst_ref[...]
    pad_seg = SEG + lax.broadcasted_iota(jnp.int32, (NTILES, pad), 0)
    zpad = jnp.zeros((NTILES, pad), jnp.int32)
    seg_ref[...] = jnp.concatenate([seg, pad_seg], axis=1)
    gb4_ref[...] = jnp.concatenate([src * C0, zpad], axis=1)
    gb8_ref[...] = jnp.concatenate([src * C12, zpad], axis=1)


def _edge_prep(src, dst, et):
    out = jax.ShapeDtypeStruct((NTILES, NBATCH * BB), jnp.int32)
    return pl.pallas_call(
        _edge_prep_body, out_shape=[out, out, out],
    )(src.reshape(NTILES, EP), dst.reshape(NTILES, EP), et.reshape(NTILES, EP))


def _make_sc_scatter(C, with_counts):
    C_sc = C // NCORES
    mesh = plsc.VectorSubcoreMesh(core_axis_name="c", subcore_axis_name="s")
    out_type = [jax.ShapeDtypeStruct((SEG, C * CHUNK), jnp.float32)]
    if with_counts:
        out_type.append(jax.ShapeDtypeStruct((SEG, CHUNK), jnp.float32))
    scratch = [
        pltpu.VMEM((PB, BB), jnp.int32),
        pltpu.VMEM((PB, BB), jnp.int32),
        pltpu.VMEM((RING, BB, CHUNK), jnp.float32),
        pltpu.VMEM((BB, CHUNK), jnp.float32),
        pltpu.VMEM_SHARED((ACC_ROWS, CHUNK), jnp.float32),
        pltpu.SemaphoreType.DMA,
        pltpu.SemaphoreType.DMA,
    ]

    @functools.partial(pl.kernel, mesh=mesh, out_type=out_type,
                       scratch_types=scratch,
                       compiler_params=pltpu.CompilerParams(
                           use_tc_tiling_on_sc=False))
    def sck(xr_hbm, seg_hbm, gb_hbm, zeros_hbm, ones_hbm, acc_hbm, *rest):
        if with_counts:
            cnt_hbm, seg2d, gidx2d, ring, zbuf, acc_sp, sem, sem_s = rest
        else:
            seg2d, gidx2d, ring, zbuf, acc_sp, sem, sem_s = rest
        core = lax.axis_index("c")
        s = lax.axis_index("s")

        pltpu.sync_copy(zeros_hbm, zbuf)

        def zero_acc():
            base = s * ZROWS
            nfull = ZROWS // BB
            tail = ZROWS - nfull * BB
            def zbody(k, carry):
                pltpu.async_copy(zbuf, acc_sp.at[pl.ds(base + k * BB, BB)],
                                 sem)
                return carry
            lax.fori_loop(0, nfull, zbody, 0)
            pltpu.async_copy(zbuf.at[pl.ds(0, tail)],
                             acc_sp.at[pl.ds(base + nfull * BB, tail)], sem)
            def zwait(k, carry):
                pltpu.make_async_copy(
                    zbuf, acc_sp.at[pl.ds(base, BB)], sem).wait()
                return carry
            lax.fori_loop(0, nfull, zwait, 0)
            pltpu.make_async_copy(
                zbuf.at[pl.ds(0, tail)],
                acc_sp.at[pl.ds(base, tail)], sem).wait()

        def drain_scatters(n):
            for _ in range(n):
                pltpu.make_async_copy(
                    ring.at[0], acc_sp.at[seg2d.at[0]], sem_s).wait()

        for cc in range(C_sc):
            c = core * C_sc + cc

            zero_acc()
            plsc.subcore_barrier()

            for p in range(NPASS):
                pltpu.sync_copy(seg_hbm.at[s, pl.ds(p * PB, PB)], seg2d)
                pltpu.sync_copy(gb_hbm.at[s, pl.ds(p * PB, PB)], gidx2d)

                def gi_body(j, carry):
                    for g in range(BB // 16):
                        sl = pl.ds(g * 16, 16)
                        gidx2d[j, sl] = gidx2d[j, sl] + c
                    return carry
                lax.fori_loop(0, PB, gi_body, 0)

                for b in range(RING - KS):
                    pltpu.make_async_copy(
                        xr_hbm.at[gidx2d.at[b]], ring.at[b], sem).start()

                def round_body(rr, carry):
                    for b in range(RING):
                        j = rr * RING + b
                        jn = j + RING - KS
                        bks = (b - KS) % RING
                        @pl.when(jnp.logical_and(j >= KS, jn < PB))
                        def _():
                            pltpu.make_async_copy(
                                ring.at[0], acc_sp.at[seg2d.at[0]],
                                sem_s).wait()
                            pltpu.make_async_copy(
                                xr_hbm.at[gidx2d.at[jn]], ring.at[bks],
                                sem).start()
                        @pl.when(j < KS)
                        def _():
                            pltpu.make_async_copy(
                                xr_hbm.at[gidx2d.at[jn]], ring.at[bks],
                                sem).start()
                        pltpu.make_async_copy(
                            xr_hbm.at[gidx2d.at[0]], ring.at[b], sem).wait()
                        pltpu.async_copy(ring.at[b], acc_sp.at[seg2d.at[j]],
                                         sem_s, add=True)
                    return carry
                lax.fori_loop(0, PB // RING, round_body, 0)
                drain_scatters(RING)
            plsc.subcore_barrier()

            pltpu.sync_copy(
                acc_sp.at[pl.ds(s * DROWS, DROWS)],
                acc_hbm.at[pl.ds(s * DROWS, DROWS),
                           pl.ds(c * CHUNK, CHUNK)])
            plsc.subcore_barrier()

        if with_counts:
            @pl.when(core == 0)
            def _():
                zero_acc()
                plsc.subcore_barrier()
                pltpu.sync_copy(ones_hbm, ring.at[0])
                for p in range(NPASS):
                    pltpu.sync_copy(seg_hbm.at[s, pl.ds(p * PB, PB)], seg2d)
                    def cgroup(g, carry):
                        for b in range(RING):
                            j = g * RING + b
                            pltpu.async_copy(ring.at[0],
                                             acc_sp.at[seg2d.at[j]],
                                             sem_s, add=True)
                        drain_scatters(RING)
                        return carry
                    lax.fori_loop(0, PB // RING, cgroup, 0)
                plsc.subcore_barrier()
                pltpu.sync_copy(acc_sp.at[pl.ds(s * DROWS, DROWS)],
                                cnt_hbm.at[pl.ds(s * DROWS, DROWS)])

    return sck


_sc_cache = {}


def _sc_scatter(C, with_counts):
    key = (C, with_counts)
    if key not in _sc_cache:
        _sc_cache[key] = _make_sc_scatter(C, with_counts)
    return _sc_cache[key]


def _wcomb_body(comp_ref, basesf_ref, out_ref):
    out_ref[...] = jnp.dot(comp_ref[...], basesf_ref[...],
                           preferred_element_type=jnp.float32)


def _wcomb(comp, basesf):
    L = basesf.shape[1]
    wblk = 2048
    return pl.pallas_call(
        _wcomb_body,
        grid=(L // wblk,),
        in_specs=[pl.BlockSpec((R, NB), lambda i: (0, 0)),
                  pl.BlockSpec((NB, wblk), lambda i: (0, i))],
        out_specs=pl.BlockSpec((R, wblk), lambda i: (0, i)),
        out_shape=jax.ShapeDtypeStruct((R, L), jnp.float32),
    )(comp, basesf)


def _layer_body(res, acc_ref, cnt_ref, h_ref, w_ref, root_ref, bias_ref,
                g_ref, b_ref, out_ref):
    cnt = cnt_ref[...][:, :, :1]
    recip = 1.0 / jnp.maximum(cnt, 1.0)
    m = acc_ref[...] * recip
    y = jnp.dot(h_ref[...], root_ref[...], preferred_element_type=jnp.float32)
    for r in range(R):
        y = y + jnp.dot(m[r], w_ref[r], preferred_element_type=jnp.float32)
    y = y + bias_ref[...]
    mu = jnp.mean(y, axis=-1, keepdims=True)
    d = y - mu
    var = jnp.mean(d * d, axis=-1, keepdims=True)
    y = d * lax.rsqrt(var + 1e-5) * g_ref[...] + b_ref[...]
    y = jnp.where(y > 0, y, jnp.exp(y) - 1.0)
    if res:
        y = y + h_ref[...]
    out_ref[...] = y


def _layer(res, acc3, cnt3, h, w3, root, bias, g, b):
    di = h.shape[1]
    vec = pl.BlockSpec((1, DH), lambda i: (0, 0))
    return pl.pallas_call(
        functools.partial(_layer_body, res),
        grid=(N // BN,),
        in_specs=[pl.BlockSpec((R, BN, di), lambda i: (0, i, 0)),
                  pl.BlockSpec((R, BN, CHUNK), lambda i: (0, i, 0)),
                  pl.BlockSpec((BN, di), lambda i: (i, 0)),
                  pl.BlockSpec((R, di, DH), lambda i: (0, 0, 0)),
                  pl.BlockSpec((di, DH), lambda i: (0, 0)),
                  vec, vec, vec],
        out_specs=pl.BlockSpec((BN, DH), lambda i: (i, 0)),
        out_shape=jax.ShapeDtypeStruct((N, DH), jnp.float32),
    )(acc3, cnt3, h, w3, root, bias, g, b)


def _final_body(h_ref, ee_ref, w1_ref, b1_ref, w2_ref, b2_ref, out_ref, sacc):
    i = pl.program_id(0)

    @pl.when(i == 0)
    def _():
        sacc[...] = jnp.zeros_like(sacc)

    sacc[0:1, :] = sacc[0:1, :] + jnp.sum(h_ref[...], axis=0, keepdims=True)

    @pl.when(i == N // BN - 1)
    def _():
        gmean = sacc[0:1, :] * (1.0 / N)
        rep = jnp.broadcast_to(gmean, (8, DH))
        att = jnp.concatenate([rep, ee_ref[...]], axis=1)
        a = jnp.dot(att, w1_ref[...], preferred_element_type=jnp.float32)
        a = jnp.maximum(a + b1_ref[...], 0.0)
        logits = jnp.dot(a, w2_ref[...], preferred_element_type=jnp.float32)
        logits = logits + b2_ref[...]
        lane = lax.broadcasted_iota(jnp.int32, (8, PADLANES), 1)
        ml = jnp.where(lane < R, logits, jnp.full_like(logits, -1e30))
        mx = jnp.max(ml, axis=-1, keepdims=True)
        ex = jnp.exp(ml - mx)
        ex = jnp.where(lane < R, ex, 0.0)
        out_ref[...] = ex / jnp.sum(ex, axis=-1, keepdims=True)


def _final(h, ee_pad, w1, b1, w2_pad, b2_pad):
    return pl.pallas_call(
        _final_body,
        grid=(N // BN,),
        in_specs=[pl.BlockSpec((BN, DH), lambda i: (i, 0)),
                  pl.BlockSpec((8, DH), lambda i: (0, 0)),
                  pl.BlockSpec((2 * DH, DH), lambda i: (0, 0)),
                  pl.BlockSpec((1, DH), lambda i: (0, 0)),
                  pl.BlockSpec((DH, PADLANES), lambda i: (0, 0)),
                  pl.BlockSpec((1, PADLANES), lambda i: (0, 0))],
        out_specs=pl.BlockSpec((8, PADLANES), lambda i: (0, 0)),
        out_shape=jax.ShapeDtypeStruct((8, PADLANES), jnp.float32),
        scratch_shapes=[pltpu.VMEM((8, DH), jnp.float32)],
    )(h, ee_pad, w1, b1, w2_pad, b2_pad)


def kernel(x, edge_index, edge_type, bases0, comp0, root0, bias0, ln_g0,
           ln_b0, bases1, comp1, root1, bias1, ln_g1, ln_b1, bases2, comp2,
           root2, bias2, ln_g2, ln_b2, edge_emb, attn_w1, attn_b1, attn_w2,
           attn_b2):
    src = edge_index[0]
    dst = edge_index[1]
    seg3, gb4, gb8 = _edge_prep(src, dst, edge_type)
    seg3 = seg3.reshape(NTILES, NBATCH, BB)
    gb4 = gb4.reshape(NTILES, NBATCH, BB)
    gb8 = gb8.reshape(NTILES, NBATCH, BB)
    zeros = jnp.zeros((BB, CHUNK), jnp.float32)
    ones = jnp.ones((BB, CHUNK), jnp.float32)

    params = [
        (bases0, comp0, root0, bias0, ln_g0, ln_b0),
        (bases1, comp1, root1, bias1, ln_g1, ln_b1),
        (bases2, comp2, root2, bias2, ln_g2, ln_b2),
    ]
    h = x
    cnt3 = None
    for l, (ba, co, ro, bi, g, b) in enumerate(params):
        di = h.shape[1]
        C = di // CHUNK
        xr = h.reshape(N * C, CHUNK)
        if l == 0:
            acc, cntw = _sc_scatter(C0, True)(xr, seg3, gb4, zeros, ones)
            cnt3 = cntw.reshape(R, N, CHUNK)
        else:
            out = _sc_scatter(C12, False)(xr, seg3, gb8, zeros, ones)
            acc = out[0] if isinstance(out, (list, tuple)) else out
        w3 = _wcomb(co, ba.reshape(NB, di * DH)).reshape(R, di, DH)
        h = _layer(l > 0, acc.reshape(R, N, di), cnt3, h, w3, ro,
                   bi.reshape(1, DH), g.reshape(1, DH), b.reshape(1, DH))

    ee_pad = jnp.concatenate(
        [edge_emb, jnp.zeros((8 - R, DH), jnp.float32)], axis=0)
    w2_pad = jnp.concatenate(
        [attn_w2, jnp.zeros((DH, PADLANES - R), jnp.float32)], axis=1)
    b2_pad = jnp.concatenate(
        [attn_b2, jnp.zeros((PADLANES - R,), jnp.float32)]).reshape(1, PADLANES)
    scores = _final(h, ee_pad, attn_w1, attn_b1.reshape(1, DH),
                    w2_pad, b2_pad)[:R, :R]
    return h, scores
